# Initial kernel scaffold; baseline (speedup 1.0000x reference)
import jax
import jax.numpy as jnp
from jax.experimental import pallas as pl


def kernel(x, w1, b1, w2, b2, w3, b3, d1, e1, d2, e2, d3, e3):
    raise NotImplementedError("write your pallas kernel here")


# speedup vs baseline: 4.1018x; 4.1018x over previous
import jax
import jax.numpy as jnp
from jax.experimental import pallas as pl
from jax.experimental.pallas import tpu as pltpu

_F32 = jnp.float32


def _shift_prev(z):
    return jnp.concatenate([jnp.zeros_like(z[:, :1]), z[:, :-1]], axis=1)


def _shift_next(z):
    return jnp.concatenate([z[:, 1:], jnp.zeros_like(z[:, :1])], axis=1)


def _chanmm(w, h):
    c, l, bt = h.shape
    y = jnp.dot(w, h.reshape(c, l * bt), preferred_element_type=_F32)
    return y.reshape(w.shape[0], l, bt)


def _maxpool2(z):
    c, l, bt = z.shape
    return jnp.max(z.reshape(c, l // 2, 2, bt), axis=2)


def _interleave(even, odd):
    c, m, bt = even.shape
    return jnp.stack([even, odd], axis=2).reshape(c, 2 * m, bt)


def _conv1d(h, ws, b):
    cout = ws.shape[0] // 3
    ya = _chanmm(ws, h)
    y0, y1, y2 = ya[:cout], ya[cout:2 * cout], ya[2 * cout:]
    return y1 + _shift_prev(y0) + _shift_next(y2) + b[:, :, None]


def _conv_transpose1d(h, ds, e):
    cout = ds.shape[0] // 4
    ya = _chanmm(ds, h)
    y0, y1, y2, y3 = (ya[k * cout:(k + 1) * cout] for k in range(4))
    even = y1 + _shift_prev(y3)
    odd = y2 + _shift_next(y0)
    return _interleave(even, odd) + e[:, :, None]


def _autoenc_body(x_ref, w1k, b1, w2s, b2, w3s, b3,
                  d1s, e1, d2s, e2, d3s, e3, out_ref):
    relu = lambda v: jnp.maximum(v, 0.0)

    h = jnp.transpose(x_ref[...])[None, :, :]

    x3 = jnp.concatenate([_shift_prev(h), h, _shift_next(h)], axis=0)
    h = relu(_chanmm(w1k[...], x3) + b1[...][:, :, None])
    h = _maxpool2(h)
    h = relu(_conv1d(h, w2s[...], b2[...]))
    h = _maxpool2(h)
    h = relu(_conv1d(h, w3s[...], b3[...]))
    h = _maxpool2(h)

    h = relu(_conv_transpose1d(h, d1s[...], e1[...]))
    h = relu(_conv_transpose1d(h, d2s[...], e2[...]))
    h = _conv_transpose1d(h, d3s[...], e3[...])

    out_ref[...] = jnp.transpose(h[0])


def _resident(a):
    nd = a.ndim
    return pl.BlockSpec(a.shape, lambda g, _nd=nd: (0,) * _nd)


def kernel(x, w1, b1, w2, b2, w3, b3, d1, e1, d2, e2, d3, e3):
    B, L, cin = x.shape
    assert cin == 1 and L % 8 == 0

    col = lambda b: jnp.asarray(b, _F32).reshape(-1, 1)

    def conv_w(w):
        t = jnp.transpose(jnp.asarray(w, _F32), (2, 0, 1))
        return t.reshape(3 * t.shape[1], t.shape[2])

    def conv_t_w(d):
        t = jnp.transpose(jnp.asarray(d, _F32), (2, 1, 0))
        return t.reshape(4 * t.shape[1], t.shape[2])

    w1k = jnp.asarray(w1, _F32)[:, 0, :]

    bt = 128
    while B % bt:
        bt //= 2
    G = B // bt

    x2 = jnp.asarray(x, _F32)[:, :, 0]
    args = (x2, w1k, col(b1), conv_w(w2), col(b2), conv_w(w3), col(b3),
            conv_t_w(d1), col(e1), conv_t_w(d2), col(e2), conv_t_w(d3), col(e3))

    in_specs = [pl.BlockSpec((bt, L), lambda g: (g, 0))]
    in_specs += [_resident(a) for a in args[1:]]
    out_specs = pl.BlockSpec((bt, L), lambda g: (g, 0))

    y = pl.pallas_call(
        _autoenc_body,
        out_shape=jax.ShapeDtypeStruct((B, L), _F32),
        grid=(G,),
        in_specs=in_specs,
        out_specs=out_specs,
        compiler_params=pltpu.CompilerParams(
            dimension_semantics=("parallel",),
            vmem_limit_bytes=100 * 2**20,
        ),
    )(*args)
    return y.reshape(B, L, 1)


# speedup vs baseline: 26.9110x; 6.5608x over previous
import jax
import jax.numpy as jnp
from jax.experimental import pallas as pl
from jax.experimental.pallas import tpu as pltpu

_F32 = jnp.float32
_B = 128


def _shift_prev(z):
    return jnp.concatenate([jnp.zeros_like(z[:, :_B]), z[:, :-_B]], axis=1)


def _shift_next(z):
    return jnp.concatenate([z[:, _B:], jnp.zeros_like(z[:, :_B])], axis=1)


def _mm(w, h):
    return jnp.dot(w, h, preferred_element_type=_F32)


def _maxpool2(z):
    n = z.shape[1]
    return jnp.concatenate(
        [jnp.maximum(z[:, i:i + _B], z[:, i + _B:i + 2 * _B])
         for i in range(0, n, 2 * _B)], axis=1)


def _interleave(even, odd):
    n = even.shape[1]
    cols = []
    for i in range(0, n, _B):
        cols.append(even[:, i:i + _B])
        cols.append(odd[:, i:i + _B])
    return jnp.concatenate(cols, axis=1)


def _conv1d(h, ws, b):
    cout = ws.shape[0] // 3
    ya = _mm(ws, h)
    y0, y1, y2 = ya[:cout], ya[cout:2 * cout], ya[2 * cout:]
    return y1 + _shift_prev(y0) + _shift_next(y2) + b


def _conv_transpose1d(h, ds, e):
    cout = ds.shape[0] // 4
    ya = _mm(ds, h)
    y0, y1, y2, y3 = (ya[k * cout:(k + 1) * cout] for k in range(4))
    even = y1 + _shift_prev(y3)
    odd = y2 + _shift_next(y0)
    return _interleave(even, odd) + e


def _autoenc_body(x_ref, w1k, b1, w2s, b2, w3s, b3,
                  d1s, e1, d2s, e2, d3s, e3, out_ref):
    relu = lambda v: jnp.maximum(v, 0.0)
    bt, L = x_ref.shape

    h = jnp.transpose(x_ref[...]).reshape(1, L * bt)

    x3 = jnp.concatenate([_shift_prev(h), h, _shift_next(h)], axis=0)
    h = relu(_mm(w1k[...], x3) + b1[...])
    h = _maxpool2(h)
    h = relu(_conv1d(h, w2s[...], b2[...]))
    h = _maxpool2(h)
    h = relu(_conv1d(h, w3s[...], b3[...]))
    h = _maxpool2(h)

    h = relu(_conv_transpose1d(h, d1s[...], e1[...]))
    h = relu(_conv_transpose1d(h, d2s[...], e2[...]))
    h = _conv_transpose1d(h, d3s[...], e3[...])

    out_ref[...] = jnp.transpose(h.reshape(L, bt))


def _resident(a):
    nd = a.ndim
    return pl.BlockSpec(a.shape, lambda g, _nd=nd: (0,) * _nd)


def kernel(x, w1, b1, w2, b2, w3, b3, d1, e1, d2, e2, d3, e3):
    B, L, cin = x.shape
    assert cin == 1 and L % 8 == 0

    col = lambda b: jnp.asarray(b, _F32).reshape(-1, 1)

    def conv_w(w):
        t = jnp.transpose(jnp.asarray(w, _F32), (2, 0, 1))
        return t.reshape(3 * t.shape[1], t.shape[2])

    def conv_t_w(d):
        t = jnp.transpose(jnp.asarray(d, _F32), (2, 1, 0))
        return t.reshape(4 * t.shape[1], t.shape[2])

    w1k = jnp.asarray(w1, _F32)[:, 0, :]

    assert B % _B == 0
    G = B // _B

    x2 = jnp.asarray(x, _F32)[:, :, 0]
    args = (x2, w1k, col(b1), conv_w(w2), col(b2), conv_w(w3), col(b3),
            conv_t_w(d1), col(e1), conv_t_w(d2), col(e2), conv_t_w(d3), col(e3))

    in_specs = [pl.BlockSpec((_B, L), lambda g: (g, 0))]
    in_specs += [_resident(a) for a in args[1:]]
    out_specs = pl.BlockSpec((_B, L), lambda g: (g, 0))

    y = pl.pallas_call(
        _autoenc_body,
        out_shape=jax.ShapeDtypeStruct((B, L), _F32),
        grid=(G,),
        in_specs=in_specs,
        out_specs=out_specs,
        compiler_params=pltpu.CompilerParams(
            dimension_semantics=("parallel",),
            vmem_limit_bytes=100 * 2**20,
        ),
    )(*args)
    return y.reshape(B, L, 1)
